# Initial kernel scaffold; baseline (speedup 1.0000x reference)
import jax
import jax.numpy as jnp
from jax.experimental import pallas as pl


def kernel(x, edge_index, edge_weight, Wl1, bl1, Wr1, Wl2, bl2, Wr2):
    raise NotImplementedError("write your pallas kernel here")


# speedup vs baseline: 5.1480x; 5.1480x over previous
import functools

import jax
import jax.numpy as jnp
from jax import lax
from jax.experimental import pallas as pl
from jax.experimental.pallas import tpu as pltpu
from jax.experimental.pallas import tpu_sc as plsc

N_NODES = 10000
N_EDGES = 320000
DIM = 128
LANES = 16

CHUNK = 128
NC, NS = 2, 16
NW = NC * NS
EPW = N_EDGES // NW
WCHUNKS = -(-EPW // CHUNK)
PAD_TOTAL = NW * WCHUNKS * CHUNK - N_EDGES

N_PAD = N_NODES + 8
NROWCH = -(-N_PAD // CHUNK)
Z_TAIL = N_PAD - (NROWCH - 1) * CHUNK
W_TAIL = N_NODES - (NROWCH - 1) * CHUNK
WCH = -(-NROWCH // NS)
H_PAD = N_NODES + LANES

RB = 400

_MESH = dict(core_axis_name="c", subcore_axis_name="s")


def _zero_rows(buf, ncols):
    zero16 = jnp.zeros((LANES,), jnp.float32)

    def zrow(i, _):
        def zcol(j, _):
            buf[i, pl.ds(j * LANES, LANES)] = zero16
            return 0
        lax.fori_loop(0, ncols // LANES, zcol, 0)
        return 0
    lax.fori_loop(0, CHUNK, zrow, 0)


def _init_shared(sh, buf, s, tail):
    def zsh(j, _):
        jj = j * NS + s

        @pl.when(jj < NROWCH - 1)
        def _():
            pltpu.sync_copy(buf, sh.at[pl.ds(jj * CHUNK, CHUNK)])

        @pl.when(jj == NROWCH - 1)
        def _():
            pltpu.sync_copy(buf.at[pl.ds(0, tail)],
                            sh.at[pl.ds(jj * CHUNK, tail)])
        return 0
    lax.fori_loop(0, WCH, zsh, 0)


def _write_shared(sh, out, c, s):
    def wout(j, _):
        jj = j * NS + s

        @pl.when(jj < NROWCH - 1)
        def _():
            pltpu.sync_copy(sh.at[pl.ds(jj * CHUNK, CHUNK)],
                            out.at[c, pl.ds(jj * CHUNK, CHUNK)])

        @pl.when(jj == NROWCH - 1)
        def _():
            pltpu.sync_copy(sh.at[pl.ds(jj * CHUNK, W_TAIL)],
                            out.at[c, pl.ds(jj * CHUNK, W_TAIL)])
        return 0
    lax.fori_loop(0, WCH, wout, 0)


def _sc_agg_body(h_hbm, src_hbm, dst_hbm, agg_out, idx_s, idx_d, rows,
                 agg_sh, sem):
    c = lax.axis_index("c")
    s = lax.axis_index("s")
    wid = s * NC + c

    _zero_rows(rows, DIM)
    _init_shared(agg_sh, rows, s, Z_TAIL)

    pltpu.sync_copy(src_hbm.at[wid], idx_s)
    pltpu.sync_copy(dst_hbm.at[wid], idx_d)

    plsc.subcore_barrier()

    def step(t, _):
        pltpu.async_copy(h_hbm.at[idx_s.at[t]], rows, sem).wait()
        pltpu.sync_copy(rows, agg_sh.at[idx_d.at[t]], add=True)
        return 0
    lax.fori_loop(0, WCHUNKS, step, 0)

    plsc.subcore_barrier()
    _write_shared(agg_sh, agg_out, c, s)


def _sc_cnt_body(dst_hbm, cnt_out, idx_d, hist):
    c = lax.axis_index("c")
    s = lax.axis_index("s")
    wid = s * NC + c

    zero16 = jnp.zeros((LANES,), jnp.float32)

    def zhist(i, _):
        hist[pl.ds(i * LANES, LANES)] = zero16
        return 0
    lax.fori_loop(0, H_PAD // LANES, zhist, 0)

    pltpu.sync_copy(dst_hbm.at[pl.ds(wid * EPW, EPW)], idx_d)

    ones16 = jnp.ones((LANES,), jnp.float32)

    def step(t, _):
        iv = idx_d[pl.ds(t * LANES, LANES)]
        plsc.addupdate_scatter(hist, [iv], ones16)
        return 0
    lax.fori_loop(0, EPW // LANES, step, 0)

    pltpu.sync_copy(hist.at[pl.ds(0, N_NODES)], cnt_out.at[wid, 0])


def _make_sc_agg():
    return pl.kernel(
        _sc_agg_body,
        out_type=jax.ShapeDtypeStruct((NC, N_NODES, DIM), jnp.float32),
        mesh=plsc.VectorSubcoreMesh(**_MESH),
        scratch_types=[
            pltpu.VMEM((WCHUNKS, CHUNK), jnp.int32),
            pltpu.VMEM((WCHUNKS, CHUNK), jnp.int32),
            pltpu.VMEM((CHUNK, DIM), jnp.float32),
            pltpu.VMEM_SHARED((N_PAD, DIM), jnp.float32),
            pltpu.SemaphoreType.DMA,
        ])


def _make_sc_cnt():
    return pl.kernel(
        _sc_cnt_body,
        out_type=jax.ShapeDtypeStruct((NW, 1, N_NODES), jnp.float32),
        mesh=plsc.VectorSubcoreMesh(**_MESH),
        compiler_params=pltpu.CompilerParams(needs_layout_passes=False),
        scratch_types=[
            pltpu.VMEM((EPW,), jnp.int32),
            pltpu.VMEM((H_PAD,), jnp.float32),
        ])


def _tc_layer_body(relu, p_ref, c_ref, x_ref, wl_ref, bl_ref, wr_ref, o_ref):
    deg = jnp.maximum(jnp.sum(c_ref[...], axis=1), 1.0)
    mean = (p_ref[0] + p_ref[1]) / deg[:, None]
    acc = jnp.dot(mean, wl_ref[...], preferred_element_type=jnp.float32)
    acc = acc + bl_ref[...]
    acc = acc + jnp.dot(x_ref[...], wr_ref[...],
                        preferred_element_type=jnp.float32)
    o_ref[...] = jnp.maximum(acc, 0.0) if relu else acc


def _tc_layer(p, cnt, x, Wl, bl, Wr, relu):
    return pl.pallas_call(
        functools.partial(_tc_layer_body, relu),
        grid=(N_NODES // RB,),
        in_specs=[
            pl.BlockSpec((NC, RB, DIM), lambda i: (0, i, 0)),
            pl.BlockSpec((RB, NW), lambda i: (i, 0)),
            pl.BlockSpec((RB, DIM), lambda i: (i, 0)),
            pl.BlockSpec((DIM, DIM), lambda i: (0, 0)),
            pl.BlockSpec((1, DIM), lambda i: (0, 0)),
            pl.BlockSpec((DIM, DIM), lambda i: (0, 0)),
        ],
        out_specs=pl.BlockSpec((RB, DIM), lambda i: (i, 0)),
        out_shape=jax.ShapeDtypeStruct((N_NODES, DIM), jnp.float32),
    )(p, cnt, x, Wl, bl.reshape(1, DIM), Wr)


def kernel(x, edge_index, edge_weight, Wl1, bl1, Wr1, Wl2, bl2, Wr2):
    del edge_weight
    src = jnp.concatenate(
        [edge_index[0], jnp.zeros((PAD_TOTAL,), jnp.int32)]
    ).reshape(NW, WCHUNKS, CHUNK)
    dst = jnp.concatenate(
        [edge_index[1], jnp.full((PAD_TOTAL,), N_NODES, jnp.int32)]
    ).reshape(NW, WCHUNKS, CHUNK)
    cnt = _make_sc_cnt()(edge_index[1]).reshape(NW, N_NODES).T
    agg1 = _make_sc_agg()(x, src, dst)
    h = _tc_layer(agg1, cnt, x, Wl1, bl1, Wr1, relu=True)
    agg2 = _make_sc_agg()(h, src, dst)
    out = _tc_layer(agg2, cnt, h, Wl2, bl2, Wr2, relu=False)
    return out
